# baseline (device time: 28280 ns/iter reference)
import jax
import jax.numpy as jnp
from jax import lax
from jax.experimental import pallas as pl
from jax.experimental.pallas import tpu as pltpu

N_DEV = 8

RS_CLIP = 108.0
AG_CLIP = 310.0


def kernel(A, B):
    m, k = A.shape
    _, n = B.shape
    chunk = m // N_DEV
    half = chunk // 2

    def silu(x):
        return x / (1.0 + jnp.exp(-x))

    def body(a_ref, b_ref, out_ref, p_ref, qbuf, rbuf, qz, zbuf,
             rs_send, rs_recv, ag_send, ag_recv):
        me = lax.axis_index("i")

        barrier = pltpu.get_barrier_semaphore()
        for o in range(1, N_DEV):
            pl.semaphore_signal(barrier, inc=1,
                                device_id=((me + o) % N_DEV,),
                                device_id_type=pl.DeviceIdType.MESH)

        b_bf16 = b_ref[...].astype(jnp.bfloat16)

        def rs_copy(src_rows, slot, peer):
            return pltpu.make_async_remote_copy(
                src_ref=qbuf.at[src_rows, :],
                dst_ref=rbuf.at[slot],
                send_sem=rs_send.at[slot],
                recv_sem=rs_recv.at[slot],
                device_id=(peer,),
                device_id_type=pl.DeviceIdType.MESH,
            )

        rs_rdmas = []
        for o in range(1, N_DEV):
            peer = (me + o) % N_DEV
            prows = pl.ds(peer * chunk, chunk)
            pc = jnp.dot(
                a_ref[prows, :].astype(jnp.bfloat16),
                b_bf16,
                preferred_element_type=jnp.float32,
            )
            qbuf[prows, :] = jnp.clip(
                jnp.round(pc * (127.0 / RS_CLIP)), -127.0, 127.0
            ).astype(jnp.int8)
            if o == 1:
                pl.semaphore_wait(barrier, N_DEV - 1)
            rdma = rs_copy(pl.ds(peer * chunk, half), o - 1, peer)
            rdma.start()
            rs_rdmas.append(rdma)
        my_rows = pl.ds(me * chunk, chunk)
        p_ref[...] = jnp.dot(
            a_ref[my_rows, :].astype(jnp.bfloat16),
            b_bf16,
            preferred_element_type=jnp.float32,
        )

        def ag_copy(half_rows, slot, peer):
            return pltpu.make_async_remote_copy(
                src_ref=qz.at[half_rows, :],
                dst_ref=zbuf.at[slot],
                send_sem=ag_send.at[slot],
                recv_sem=ag_recv.at[slot],
                device_id=(peer,),
                device_id_type=pl.DeviceIdType.MESH,
            )

        ag_rdmas = []
        rs_bot_rdmas = []

        for h, rdmas, base in ((0, rs_rdmas, 0), (1, rs_bot_rdmas, 7)):
            if h == 1:
                for o in range(1, N_DEV):
                    peer = (me + o) % N_DEV
                    rdma = rs_copy(pl.ds(peer * chunk + half, half),
                                   6 + o, peer)
                    rdma.start()
                    rs_bot_rdmas.append(rdma)
            rdmas[0].wait_recv()
            acc = rbuf[base, :, :].astype(jnp.float32)
            for o in range(2, N_DEV):
                rdmas[o - 1].wait_recv()
                acc = acc + rbuf[base + o - 1, :, :].astype(jnp.float32)
            hrows = pl.ds(h * half, half)
            z = p_ref[hrows, :] + acc * (RS_CLIP / 127.0)
            qz[hrows, :] = jnp.clip(
                jnp.round(z * (127.0 / AG_CLIP)), -127.0, 127.0
            ).astype(jnp.int8)
            for o in range(1, N_DEV):
                peer = (me + o) % N_DEV
                rdma = ag_copy(pl.ds(h * half, half), base + o - 1, peer)
                rdma.start()
                ag_rdmas.append(rdma)
            out_ref[pl.ds(me * chunk + h * half, half), :] = (
                silu(z).astype(out_ref.dtype))

        for h in (0, 1):
            for o in range(1, N_DEV):
                slot = 7 * h + o - 1
                ag_rdmas[slot].wait_recv()
                sender = (me - o) % N_DEV
                srows = pl.ds(sender * chunk + h * half, half)
                zr = zbuf[slot, :, :].astype(jnp.float32) * (AG_CLIP / 127.0)
                out_ref[srows, :] = silu(zr).astype(out_ref.dtype)

        for r in rs_rdmas + rs_bot_rdmas + ag_rdmas:
            r.wait_send()

    n_slots = 2 * (N_DEV - 1)
    return pl.pallas_call(
        body,
        out_shape=jax.ShapeDtypeStruct((m, n), jnp.bfloat16),
        in_specs=[
            pl.BlockSpec(memory_space=pltpu.VMEM),
            pl.BlockSpec(memory_space=pltpu.VMEM),
        ],
        out_specs=pl.BlockSpec(memory_space=pltpu.VMEM),
        scratch_shapes=[
            pltpu.VMEM((chunk, n), jnp.float32),
            pltpu.VMEM((m, n), jnp.int8),
            pltpu.VMEM((n_slots, half, n), jnp.int8),
            pltpu.VMEM((chunk, n), jnp.int8),
            pltpu.VMEM((n_slots, half, n), jnp.int8),
            pltpu.SemaphoreType.DMA((n_slots,)),
            pltpu.SemaphoreType.DMA((n_slots,)),
            pltpu.SemaphoreType.DMA((n_slots,)),
            pltpu.SemaphoreType.DMA((n_slots,)),
        ],
        compiler_params=pltpu.CompilerParams(collective_id=0),
    )(A, B)


# device time: 23111 ns/iter; 1.2237x vs baseline; 1.2237x over previous
import jax
import jax.numpy as jnp
from jax import lax
from jax.experimental import pallas as pl
from jax.experimental.pallas import tpu as pltpu

N_DEV = 8

RS_CLIP = 108.0
AG_CLIP = 310.0


def kernel(A, B):
    m, k = A.shape
    _, n = B.shape
    chunk = m // N_DEV
    half = chunk // 2

    def silu(x):
        return x / (1.0 + jnp.exp(-x))

    def body(a_ref, b_ref, out_ref, p_ref, qbuf, rbuf, qz, zbuf,
             rs_send, rs_recv, ag_send, ag_recv):
        me = lax.axis_index("i")

        barrier = pltpu.get_barrier_semaphore()
        for o in range(1, N_DEV):
            pl.semaphore_signal(barrier, inc=1,
                                device_id=((me + o) % N_DEV,),
                                device_id_type=pl.DeviceIdType.MESH)

        b_bf16 = b_ref[...].astype(jnp.bfloat16)

        def rs_copy(src_rows, slot, peer):
            return pltpu.make_async_remote_copy(
                src_ref=qbuf.at[src_rows, :],
                dst_ref=rbuf.at[slot],
                send_sem=rs_send.at[slot],
                recv_sem=rs_recv.at[slot],
                device_id=(peer,),
                device_id_type=pl.DeviceIdType.MESH,
            )

        rs_rdmas = []
        for o in range(1, N_DEV):
            peer = (me + o) % N_DEV
            prows = pl.ds(peer * chunk, chunk)
            pc = jnp.dot(
                a_ref[prows, :].astype(jnp.bfloat16),
                b_bf16,
                preferred_element_type=jnp.float32,
            )
            qbuf[prows, :] = jnp.clip(
                jnp.round(pc * (127.0 / RS_CLIP)), -127.0, 127.0
            ).astype(jnp.int8)
            if o == 1:
                pl.semaphore_wait(barrier, N_DEV - 1)
            rdma = rs_copy(pl.ds(peer * chunk, half), o - 1, peer)
            rdma.start()
            rs_rdmas.append(rdma)
        rs_bot_rdmas = []
        for o in range(1, N_DEV):
            peer = (me + o) % N_DEV
            rdma = rs_copy(pl.ds(peer * chunk + half, half), 6 + o, peer)
            rdma.start()
            rs_bot_rdmas.append(rdma)

        my_rows = pl.ds(me * chunk, chunk)
        p_ref[...] = jnp.dot(
            a_ref[my_rows, :].astype(jnp.bfloat16),
            b_bf16,
            preferred_element_type=jnp.float32,
        )

        def ag_copy(half_rows, slot, peer):
            return pltpu.make_async_remote_copy(
                src_ref=qz.at[half_rows, :],
                dst_ref=zbuf.at[slot],
                send_sem=ag_send.at[slot],
                recv_sem=ag_recv.at[slot],
                device_id=(peer,),
                device_id_type=pl.DeviceIdType.MESH,
            )

        ag_rdmas = []

        for h, rdmas, base in ((0, rs_rdmas, 0), (1, rs_bot_rdmas, 7)):
            rdmas[0].wait_recv()
            acc = rbuf[base, :, :].astype(jnp.float32)
            for o in range(2, N_DEV):
                rdmas[o - 1].wait_recv()
                acc = acc + rbuf[base + o - 1, :, :].astype(jnp.float32)
            hrows = pl.ds(h * half, half)
            z = p_ref[hrows, :] + acc * (RS_CLIP / 127.0)
            qz[hrows, :] = jnp.clip(
                jnp.round(z * (127.0 / AG_CLIP)), -127.0, 127.0
            ).astype(jnp.int8)
            for o in range(1, N_DEV):
                peer = (me + o) % N_DEV
                rdma = ag_copy(pl.ds(h * half, half), base + o - 1, peer)
                rdma.start()
                ag_rdmas.append(rdma)
            out_ref[pl.ds(me * chunk + h * half, half), :] = (
                silu(z).astype(out_ref.dtype))

        for h in (0, 1):
            for o in range(1, N_DEV):
                slot = 7 * h + o - 1
                ag_rdmas[slot].wait_recv()
                sender = (me - o) % N_DEV
                srows = pl.ds(sender * chunk + h * half, half)
                zr = zbuf[slot, :, :].astype(jnp.float32) * (AG_CLIP / 127.0)
                out_ref[srows, :] = silu(zr).astype(out_ref.dtype)

        for r in rs_rdmas + rs_bot_rdmas + ag_rdmas:
            r.wait_send()

    n_slots = 2 * (N_DEV - 1)
    return pl.pallas_call(
        body,
        out_shape=jax.ShapeDtypeStruct((m, n), jnp.bfloat16),
        in_specs=[
            pl.BlockSpec(memory_space=pltpu.VMEM),
            pl.BlockSpec(memory_space=pltpu.VMEM),
        ],
        out_specs=pl.BlockSpec(memory_space=pltpu.VMEM),
        scratch_shapes=[
            pltpu.VMEM((chunk, n), jnp.float32),
            pltpu.VMEM((m, n), jnp.int8),
            pltpu.VMEM((n_slots, half, n), jnp.int8),
            pltpu.VMEM((chunk, n), jnp.int8),
            pltpu.VMEM((n_slots, half, n), jnp.int8),
            pltpu.SemaphoreType.DMA((n_slots,)),
            pltpu.SemaphoreType.DMA((n_slots,)),
            pltpu.SemaphoreType.DMA((n_slots,)),
            pltpu.SemaphoreType.DMA((n_slots,)),
        ],
        compiler_params=pltpu.CompilerParams(collective_id=0),
    )(A, B)
